# baseline (speedup 1.0000x reference)
import functools

import jax
import jax.numpy as jnp
from jax import lax
from jax.experimental import pallas as pl
from jax.experimental.pallas import tpu as pltpu
from jax.experimental.pallas import tpu_sc as plsc

_B = 16384
_NTAAL = 120


def _sc_gather_embeddings(mood, raga, taal, fused_table, raga_table):
    info = plsc.get_sparse_core_info()
    nc, ns = info.num_cores, info.num_subcores
    nw = nc * ns
    bpw = _B // nw
    ch = 128
    nch = bpw // ch
    lanes = info.num_lanes

    mesh = plsc.VectorSubcoreMesh(core_axis_name="c", subcore_axis_name="s")

    @functools.partial(
        pl.kernel,
        out_type=(
            jax.ShapeDtypeStruct((_B, 128), jnp.float32),
            jax.ShapeDtypeStruct((_B, 128), jnp.float32),
        ),
        mesh=mesh,
        scratch_types=[
            pltpu.VMEM((bpw,), jnp.int32),
            pltpu.VMEM((bpw,), jnp.int32),
            pltpu.VMEM((bpw,), jnp.int32),
            pltpu.VMEM((bpw,), jnp.int32),
            pltpu.VMEM((2, ch, 128), jnp.float32),
            pltpu.VMEM((2, ch, 128), jnp.float32),
            pltpu.SemaphoreType.DMA,
            pltpu.SemaphoreType.DMA,
            pltpu.SemaphoreType.DMA,
            pltpu.SemaphoreType.DMA,
            pltpu.SemaphoreType.DMA,
            pltpu.SemaphoreType.DMA,
            pltpu.SemaphoreType.DMA,
            pltpu.SemaphoreType.DMA,
        ],
    )
    def gather_kernel(mood_h, raga_h, taal_h, ft_h, rt_h,
                      of_h, or_h,
                      mi, ti, ri, fi, fbuf, rbuf,
                      gf0, gf1, gr0, gr1, wf0, wf1, wr0, wr1):
        wid = lax.axis_index("s") * nc + lax.axis_index("c")
        base = wid * bpw

        pltpu.sync_copy(mood_h.at[pl.ds(base, bpw)], mi)
        pltpu.sync_copy(taal_h.at[pl.ds(base, bpw)], ti)
        pltpu.sync_copy(raga_h.at[pl.ds(base, bpw)], ri)

        def fuse(k, carry):
            s = pl.ds(k * lanes, lanes)
            fi[s] = mi[s] * _NTAAL + ti[s]
            return carry

        lax.fori_loop(0, bpw // lanes, fuse, 0)

        gf = (gf0, gf1)
        gr = (gr0, gr1)
        wf = (wf0, wf1)
        wr = (wr0, wr1)

        def start_gather(i, slot):
            s = pl.ds(i * ch, ch)
            cf = pltpu.async_copy(ft_h.at[fi.at[s]], fbuf.at[slot], gf[slot])
            cr = pltpu.async_copy(rt_h.at[ri.at[s]], rbuf.at[slot], gr[slot])
            return cf, cr

        pend_g = start_gather(0, 0)
        pend_w = [None, None]
        for i in range(nch):
            slot = i % 2
            nxt = (i + 1) % 2
            if i + 1 < nch:
                if pend_w[nxt] is not None:
                    pend_w[nxt][0].wait()
                    pend_w[nxt][1].wait()
                    pend_w[nxt] = None
                next_g = start_gather(i + 1, nxt)
            pend_g[0].wait()
            pend_g[1].wait()
            off = base + i * ch
            cwf = pltpu.async_copy(fbuf.at[slot], of_h.at[pl.ds(off, ch)],
                                   wf[slot])
            cwr = pltpu.async_copy(rbuf.at[slot], or_h.at[pl.ds(off, ch)],
                                   wr[slot])
            pend_w[slot] = (cwf, cwr)
            if i + 1 < nch:
                pend_g = next_g
        for pw in pend_w:
            if pw is not None:
                pw[0].wait()
                pw[1].wait()

    return gather_kernel(mood, raga, taal, fused_table, raga_table)


_BLK = 4096


def _tc_body(fe_r, re_r, t_r, d_r, wt_r, bt_r, wd_r, bd_r,
             wp_r, bp_r, g_r, b_r, o_r):
    f32 = jnp.float32
    bf16 = jnp.bfloat16
    w_fused = jnp.concatenate(
        [wp_r[0:64, :], wp_r[192:256, :]], axis=0).astype(bf16)
    w_raga = wp_r[64:192, :].astype(bf16)
    w_tmp = wp_r[256:288, :]
    w_dur = wp_r[288:320, :]

    temb = t_r[...] * wt_r[...] + bt_r[...]
    demb = d_r[...] * wd_r[...] + bd_r[...]

    h = jnp.dot(fe_r[...].astype(bf16), w_fused, preferred_element_type=f32)
    h += jnp.dot(re_r[...].astype(bf16), w_raga, preferred_element_type=f32)
    h += jnp.dot(temb, w_tmp, preferred_element_type=f32)
    h += jnp.dot(demb, w_dur, preferred_element_type=f32)
    h += bp_r[...]

    h = 0.5 * h * (1.0 + lax.erf(h * 0.7071067811865476))

    mean = jnp.mean(h, axis=-1, keepdims=True)
    cent = h - mean
    var = jnp.mean(cent * cent, axis=-1, keepdims=True)
    inv = lax.rsqrt(var + 1e-5)
    o_r[...] = cent * inv * g_r[...] + b_r[...]


def _tc_mlp_ln(fe, re_, tempo, dur, W_tempo, b_tempo, W_dur, b_dur,
               W_proj, b_proj, ln_gamma, ln_beta):
    grid = (_B // _BLK,)
    row = lambda shp: pl.BlockSpec(shp, lambda i: (0, 0))
    blk = lambda w: pl.BlockSpec((_BLK, w), lambda i: (i, 0))
    return pl.pallas_call(
        _tc_body,
        grid=grid,
        in_specs=[
            blk(128), blk(128), blk(1), blk(1),
            row((1, 32)), row((1, 32)), row((1, 32)), row((1, 32)),
            row((320, 512)), row((1, 512)), row((1, 512)), row((1, 512)),
        ],
        out_specs=blk(512),
        out_shape=jax.ShapeDtypeStruct((_B, 512), jnp.float32),
    )(fe, re_, tempo, dur, W_tempo, b_tempo, W_dur, b_dur,
      W_proj, b_proj, ln_gamma, ln_beta)


def kernel(mood, raga, taal, tempo, duration, mood_table, raga_table,
           taal_table, W_tempo, b_tempo, W_dur, b_dur, W_proj, b_proj,
           ln_gamma, ln_beta):
    mood = mood.astype(jnp.int32)
    raga = raga.astype(jnp.int32)
    taal = taal.astype(jnp.int32)
    fused_table = jnp.concatenate(
        [jnp.repeat(mood_table, _NTAAL, axis=0),
         jnp.tile(taal_table, (mood_table.shape[0], 1))], axis=1)
    fe, re_ = _sc_gather_embeddings(mood, raga, taal, fused_table, raga_table)
    return _tc_mlp_ln(
        fe, re_,
        tempo.reshape(_B, 1), duration.reshape(_B, 1),
        W_tempo, b_tempo.reshape(1, 32), W_dur, b_dur.reshape(1, 32),
        W_proj, b_proj.reshape(1, 512),
        ln_gamma.reshape(1, 512), ln_beta.reshape(1, 512))


# speedup vs baseline: 1.0775x; 1.0775x over previous
import functools

import jax
import jax.numpy as jnp
from jax import lax
from jax.experimental import pallas as pl
from jax.experimental.pallas import tpu as pltpu
from jax.experimental.pallas import tpu_sc as plsc

_B = 16384
_NTAAL = 120


def _sc_gather_embeddings(mood, raga, taal, fused_table, raga_table):
    info = plsc.get_sparse_core_info()
    nc, ns = info.num_cores, info.num_subcores
    nw = nc * ns
    bpw = _B // nw
    ch = 128
    nch = bpw // ch
    lanes = info.num_lanes

    mesh = plsc.VectorSubcoreMesh(core_axis_name="c", subcore_axis_name="s")

    @functools.partial(
        pl.kernel,
        out_type=(
            jax.ShapeDtypeStruct((_B, 128), jnp.float32),
            jax.ShapeDtypeStruct((_B, 128), jnp.float32),
        ),
        mesh=mesh,
        scratch_types=[
            pltpu.VMEM((bpw,), jnp.int32),
            pltpu.VMEM((bpw,), jnp.int32),
            pltpu.VMEM((bpw,), jnp.int32),
            pltpu.VMEM((bpw,), jnp.int32),
            pltpu.VMEM((2, ch, 128), jnp.float32),
            pltpu.VMEM((2, ch, 128), jnp.float32),
            pltpu.SemaphoreType.DMA,
            pltpu.SemaphoreType.DMA,
            pltpu.SemaphoreType.DMA,
            pltpu.SemaphoreType.DMA,
            pltpu.SemaphoreType.DMA,
            pltpu.SemaphoreType.DMA,
            pltpu.SemaphoreType.DMA,
            pltpu.SemaphoreType.DMA,
        ],
    )
    def gather_kernel(mood_h, raga_h, taal_h, ft_h, rt_h,
                      of_h, or_h,
                      mi, ti, ri, fi, fbuf, rbuf,
                      gf0, gf1, gr0, gr1, wf0, wf1, wr0, wr1):
        wid = lax.axis_index("s") * nc + lax.axis_index("c")
        base = wid * bpw

        c_mi = pltpu.async_copy(mood_h.at[pl.ds(base, bpw)], mi, gf0)
        c_ti = pltpu.async_copy(taal_h.at[pl.ds(base, bpw)], ti, gf1)
        c_ri = pltpu.async_copy(raga_h.at[pl.ds(base, bpw)], ri, gr0)
        c_mi.wait()
        c_ti.wait()
        c_ri.wait()

        def fuse(k, carry):
            s = pl.ds(k * lanes, lanes)
            fi[s] = mi[s] * _NTAAL + ti[s]
            return carry

        lax.fori_loop(0, bpw // lanes, fuse, 0)

        gf = (gf0, gf1)
        gr = (gr0, gr1)
        wf = (wf0, wf1)
        wr = (wr0, wr1)

        def start_gather(i, slot):
            s = pl.ds(i * ch, ch)
            cf = pltpu.async_copy(ft_h.at[fi.at[s]], fbuf.at[slot], gf[slot])
            cr = pltpu.async_copy(rt_h.at[ri.at[s]], rbuf.at[slot], gr[slot])
            return cf, cr

        pend_g = start_gather(0, 0)
        pend_w = [None, None]
        for i in range(nch):
            slot = i % 2
            nxt = (i + 1) % 2
            if i + 1 < nch:
                if pend_w[nxt] is not None:
                    pend_w[nxt][0].wait()
                    pend_w[nxt][1].wait()
                    pend_w[nxt] = None
                next_g = start_gather(i + 1, nxt)
            pend_g[0].wait()
            pend_g[1].wait()
            off = base + i * ch
            cwf = pltpu.async_copy(fbuf.at[slot], of_h.at[pl.ds(off, ch)],
                                   wf[slot])
            cwr = pltpu.async_copy(rbuf.at[slot], or_h.at[pl.ds(off, ch)],
                                   wr[slot])
            pend_w[slot] = (cwf, cwr)
            if i + 1 < nch:
                pend_g = next_g
        for pw in pend_w:
            if pw is not None:
                pw[0].wait()
                pw[1].wait()

    return gather_kernel(mood, raga, taal, fused_table, raga_table)


_BLK = 2048


def _tc_body(fe_r, re_r, t_r, d_r, vt_r, vd_r, bias_r,
             wp_r, g_r, b_r, o_r):
    f32 = jnp.float32
    bf16 = jnp.bfloat16
    w_fused = jnp.concatenate(
        [wp_r[0:64, :], wp_r[192:256, :]], axis=0).astype(bf16)
    w_raga = wp_r[64:192, :].astype(bf16)

    h = jnp.dot(fe_r[...].astype(bf16), w_fused, preferred_element_type=f32)
    h += jnp.dot(re_r[...].astype(bf16), w_raga, preferred_element_type=f32)
    h += t_r[...] * vt_r[...] + bias_r[...]
    h += d_r[...] * vd_r[...]

    h = 0.5 * h * (1.0 + lax.erf(h * 0.7071067811865476))

    mean = jnp.mean(h, axis=-1, keepdims=True)
    cent = h - mean
    var = jnp.mean(cent * cent, axis=-1, keepdims=True)
    inv = lax.rsqrt(var + 1e-5)
    o_r[...] = cent * inv * g_r[...] + b_r[...]


def _tc_mlp_ln(fe, re_, tempo, dur, v_tempo, v_dur, bias,
               W_proj, ln_gamma, ln_beta):
    grid = (_B // _BLK,)
    row = lambda shp: pl.BlockSpec(shp, lambda i: (0, 0))
    blk = lambda w: pl.BlockSpec((_BLK, w), lambda i: (i, 0))
    return pl.pallas_call(
        _tc_body,
        grid=grid,
        in_specs=[
            blk(128), blk(128), blk(1), blk(1),
            row((1, 512)), row((1, 512)), row((1, 512)),
            row((320, 512)), row((1, 512)), row((1, 512)),
        ],
        out_specs=blk(512),
        out_shape=jax.ShapeDtypeStruct((_B, 512), jnp.float32),
    )(fe, re_, tempo, dur, v_tempo, v_dur, bias,
      W_proj, ln_gamma, ln_beta)


def kernel(mood, raga, taal, tempo, duration, mood_table, raga_table,
           taal_table, W_tempo, b_tempo, W_dur, b_dur, W_proj, b_proj,
           ln_gamma, ln_beta):
    mood = mood.astype(jnp.int32)
    raga = raga.astype(jnp.int32)
    taal = taal.astype(jnp.int32)
    fused_table = jnp.concatenate(
        [jnp.repeat(mood_table, _NTAAL, axis=0),
         jnp.tile(taal_table, (mood_table.shape[0], 1))], axis=1)
    fe, re_ = _sc_gather_embeddings(mood, raga, taal, fused_table, raga_table)
    w_tmp = W_proj[256:288, :]
    w_dur = W_proj[288:320, :]
    v_tempo = (W_tempo @ w_tmp).reshape(1, 512)
    v_dur = (W_dur @ w_dur).reshape(1, 512)
    bias = (b_proj + b_tempo @ w_tmp + b_dur @ w_dur).reshape(1, 512)
    return _tc_mlp_ln(
        fe, re_,
        tempo.reshape(_B, 1), duration.reshape(_B, 1),
        v_tempo, v_dur, bias,
        W_proj, ln_gamma.reshape(1, 512), ln_beta.reshape(1, 512))


# speedup vs baseline: 1.0856x; 1.0075x over previous
import functools

import jax
import jax.numpy as jnp
from jax import lax
from jax.experimental import pallas as pl
from jax.experimental.pallas import tpu as pltpu
from jax.experimental.pallas import tpu_sc as plsc

_B = 16384
_NTAAL = 120


def _sc_gather_embeddings(mood, raga, taal, fused_table, raga_table):
    info = plsc.get_sparse_core_info()
    nc, ns = info.num_cores, info.num_subcores
    nw = nc * ns
    bpw = _B // nw
    ch = 128
    nch = bpw // ch
    lanes = info.num_lanes

    mesh = plsc.VectorSubcoreMesh(core_axis_name="c", subcore_axis_name="s")

    @functools.partial(
        pl.kernel,
        out_type=(
            jax.ShapeDtypeStruct((_B, 128), jnp.float32),
            jax.ShapeDtypeStruct((_B, 128), jnp.float32),
        ),
        mesh=mesh,
        scratch_types=[
            pltpu.VMEM((bpw,), jnp.int32),
            pltpu.VMEM((bpw,), jnp.int32),
            pltpu.VMEM((bpw,), jnp.int32),
            pltpu.VMEM((bpw,), jnp.int32),
            pltpu.VMEM((2, ch, 128), jnp.float32),
            pltpu.VMEM((2, ch, 128), jnp.float32),
            pltpu.SemaphoreType.DMA,
            pltpu.SemaphoreType.DMA,
            pltpu.SemaphoreType.DMA,
            pltpu.SemaphoreType.DMA,
            pltpu.SemaphoreType.DMA,
            pltpu.SemaphoreType.DMA,
            pltpu.SemaphoreType.DMA,
            pltpu.SemaphoreType.DMA,
        ],
    )
    def gather_kernel(mood_h, raga_h, taal_h, ft_h, rt_h,
                      of_h, or_h,
                      mi, ti, ri, fi, fbuf, rbuf,
                      gf0, gf1, gr0, gr1, wf0, wf1, wr0, wr1):
        wid = lax.axis_index("s") * nc + lax.axis_index("c")
        base = wid * bpw

        c_mi = pltpu.async_copy(mood_h.at[pl.ds(base, bpw)], mi, gf0)
        c_ti = pltpu.async_copy(taal_h.at[pl.ds(base, bpw)], ti, gf1)
        c_ri = pltpu.async_copy(raga_h.at[pl.ds(base, bpw)], ri, gr0)
        c_mi.wait()
        c_ti.wait()
        c_ri.wait()

        def fuse(k, carry):
            s = pl.ds(k * lanes, lanes)
            fi[s] = mi[s] * _NTAAL + ti[s]
            return carry

        lax.fori_loop(0, bpw // lanes, fuse, 0)

        gf = (gf0, gf1)
        gr = (gr0, gr1)
        wf = (wf0, wf1)
        wr = (wr0, wr1)

        def start_gather(i, slot):
            s = pl.ds(i * ch, ch)
            cf = pltpu.async_copy(ft_h.at[fi.at[s]], fbuf.at[slot], gf[slot])
            cr = pltpu.async_copy(rt_h.at[ri.at[s]], rbuf.at[slot], gr[slot])
            return cf, cr

        pend_g = start_gather(0, 0)
        pend_w = [None, None]
        for i in range(nch):
            slot = i % 2
            nxt = (i + 1) % 2
            if i + 1 < nch:
                if pend_w[nxt] is not None:
                    pend_w[nxt][0].wait()
                    pend_w[nxt][1].wait()
                    pend_w[nxt] = None
                next_g = start_gather(i + 1, nxt)
            pend_g[0].wait()
            pend_g[1].wait()
            off = base + i * ch
            cwf = pltpu.async_copy(fbuf.at[slot], of_h.at[pl.ds(off, ch)],
                                   wf[slot])
            cwr = pltpu.async_copy(rbuf.at[slot], or_h.at[pl.ds(off, ch)],
                                   wr[slot])
            pend_w[slot] = (cwf, cwr)
            if i + 1 < nch:
                pend_g = next_g
        for pw in pend_w:
            if pw is not None:
                pw[0].wait()
                pw[1].wait()

    return gather_kernel(mood, raga, taal, fused_table, raga_table)


_BLK = 2048


def _tc_body(fe_r, re_r, t_r, d_r, vt_r, vd_r, bias_r,
             wp_r, g_r, b_r, o_r):
    f32 = jnp.float32
    bf16 = jnp.bfloat16
    w_fused = jnp.concatenate(
        [wp_r[0:64, :], wp_r[192:256, :]], axis=0).astype(bf16)
    w_raga = wp_r[64:192, :].astype(bf16)

    h = jnp.dot(fe_r[...].astype(bf16), w_fused, preferred_element_type=f32)
    h += jnp.dot(re_r[...].astype(bf16), w_raga, preferred_element_type=f32)
    h += (t_r[...] * vt_r[...] + d_r[...] * vd_r[...]) + bias_r[...]

    h = 0.5 * h * (1.0 + lax.erf(h * 0.7071067811865476))

    mean = jnp.mean(h, axis=-1, keepdims=True)
    var = jnp.mean(h * h, axis=-1, keepdims=True) - mean * mean
    inv = lax.rsqrt(var + 1e-5)
    o_r[...] = (h - mean) * inv * g_r[...] + b_r[...]


def _tc_mlp_ln(fe, re_, tempo, dur, v_tempo, v_dur, bias,
               W_proj, ln_gamma, ln_beta):
    grid = (_B // _BLK,)
    row = lambda shp: pl.BlockSpec(shp, lambda i: (0, 0))
    blk = lambda w: pl.BlockSpec((_BLK, w), lambda i: (i, 0))
    return pl.pallas_call(
        _tc_body,
        grid=grid,
        in_specs=[
            blk(128), blk(128), blk(1), blk(1),
            row((1, 512)), row((1, 512)), row((1, 512)),
            row((320, 512)), row((1, 512)), row((1, 512)),
        ],
        out_specs=blk(512),
        out_shape=jax.ShapeDtypeStruct((_B, 512), jnp.float32),
    )(fe, re_, tempo, dur, v_tempo, v_dur, bias,
      W_proj, ln_gamma, ln_beta)


def kernel(mood, raga, taal, tempo, duration, mood_table, raga_table,
           taal_table, W_tempo, b_tempo, W_dur, b_dur, W_proj, b_proj,
           ln_gamma, ln_beta):
    mood = mood.astype(jnp.int32)
    raga = raga.astype(jnp.int32)
    taal = taal.astype(jnp.int32)
    fused_table = jnp.concatenate(
        [jnp.repeat(mood_table, _NTAAL, axis=0),
         jnp.tile(taal_table, (mood_table.shape[0], 1))], axis=1)
    fe, re_ = _sc_gather_embeddings(mood, raga, taal, fused_table, raga_table)
    w_tmp = W_proj[256:288, :]
    w_dur = W_proj[288:320, :]
    v_tempo = (W_tempo @ w_tmp).reshape(1, 512)
    v_dur = (W_dur @ w_dur).reshape(1, 512)
    bias = (b_proj + b_tempo @ w_tmp + b_dur @ w_dur).reshape(1, 512)
    return _tc_mlp_ln(
        fe, re_,
        tempo.reshape(_B, 1), duration.reshape(_B, 1),
        v_tempo, v_dur, bias,
        W_proj, ln_gamma.reshape(1, 512), ln_beta.reshape(1, 512))
